# baseline (speedup 1.0000x reference)
import functools

import jax
import jax.numpy as jnp
from jax import lax
from jax.experimental import pallas as pl
from jax.experimental.pallas import tpu as pltpu
from jax.experimental.pallas import tpu_sc as plsc

_K = 512
_NC, _NS, _L = 2, 16, 16
_NW = _NC * _NS
_RPW = _K // _NW


def _scores_body(h_ref, w_ref, s_ref):
    h = h_ref[0]
    w = w_ref[...]
    nrm = jnp.sqrt(jnp.sum(w * w))
    s = jnp.dot(h, w, preferred_element_type=jnp.float32)
    s_ref[0] = s / nrm


def _select_body(sr_ref, sc_ref, h_ref, hret_ref, idxl_ref, idxg_ref):
    b = pl.program_id(0)
    s_row = sr_ref[0]
    s_col = sc_ref[0]
    h = h_ref[0]
    n = s_col.shape[0]

    cb = 512
    rank_chunks = []
    for j0 in range(0, n, cb):
        s_rj = lax.slice(s_row, (0, j0), (1, j0 + cb))
        ii = lax.broadcasted_iota(jnp.int32, (n, cb), 0)
        jj = lax.broadcasted_iota(jnp.int32, (n, cb), 1) + j0
        beats = (s_col > s_rj) | ((s_col == s_rj) & (ii < jj))
        rank_chunks.append(
            jnp.sum(beats.astype(jnp.int32), axis=0, keepdims=True))
    ranks = jnp.concatenate(rank_chunks, axis=1)

    rr = lax.broadcasted_iota(jnp.int32, (_K, 1), 0)
    mask = ranks == rr
    jj_row = lax.broadcasted_iota(jnp.int32, (1, n), 1)
    idx = jnp.sum(jnp.where(mask, jj_row, 0), axis=1, keepdims=True)
    ssel = jnp.sum(jnp.where(mask, s_row, 0.0), axis=1, keepdims=True)
    gate = jax.nn.sigmoid(ssel)
    pf = mask.astype(jnp.float32)
    hret = jnp.dot(pf, h, preferred_element_type=jnp.float32,
                   precision=jax.lax.Precision.HIGHEST)
    hret_ref[0] = hret * gate
    idxl_ref[0] = idx
    idxg_ref[0] = idx + b * n


def _make_sc_gather(B, N):
    mesh = plsc.VectorSubcoreMesh(core_axis_name="c", subcore_axis_name="s")

    @functools.partial(
        pl.kernel,
        mesh=mesh,
        out_type=jax.ShapeDtypeStruct((B * _K, _K), jnp.float32),
        scratch_types=[
            pltpu.VMEM((_RPW,), jnp.int32),
            pltpu.VMEM((_K,), jnp.int32),
            pltpu.VMEM((_RPW, N), jnp.float32),
            pltpu.VMEM((_RPW, _K), jnp.float32),
            pltpu.SemaphoreType.DMA,
        ],
        compiler_params=pltpu.CompilerParams(use_tc_tiling_on_sc=False,
                                             needs_layout_passes=False),
    )
    def gather_kernel(gflat, idxg, idxl, out, rowidx_v, colidx_v, rows_v,
                      outb_v, sem):
        wid = lax.axis_index("s") * _NC + lax.axis_index("c")
        for b in range(B):
            base = b * _K + wid * _RPW
            pltpu.sync_copy(idxg.at[pl.ds(base, _RPW)], rowidx_v)
            pltpu.sync_copy(idxl.at[pl.ds(b * _K, _K)], colidx_v)
            pltpu.async_copy(gflat.at[rowidx_v], rows_v, sem).wait()

            def row_body(r, carry):
                rsplat = jnp.broadcast_to(r, (_L,)).astype(jnp.int32)
                for t in range(_K // _L):
                    cols = colidx_v[pl.ds(t * _L, _L)]
                    vals = plsc.load_gather(rows_v, [rsplat, cols])
                    outb_v[r, pl.ds(t * _L, _L)] = vals
                return carry

            lax.fori_loop(0, _RPW, row_body, 0)
            pltpu.sync_copy(outb_v, out.at[pl.ds(base, _RPW)])

    return gather_kernel


def kernel(H, G, score_weight):
    B, N, F = H.shape
    s_col3 = pl.pallas_call(
        _scores_body,
        grid=(B,),
        in_specs=[pl.BlockSpec((1, N, F), lambda b: (b, 0, 0)),
                  pl.BlockSpec((F, 1), lambda b: (0, 0))],
        out_specs=pl.BlockSpec((1, N, 1), lambda b: (b, 0, 0)),
        out_shape=jax.ShapeDtypeStruct((B, N, 1), jnp.float32),
    )(H, score_weight)
    s_row3 = s_col3.reshape(B, 1, N)

    hret, idxl3, idxg3 = pl.pallas_call(
        _select_body,
        grid=(B,),
        in_specs=[pl.BlockSpec((1, 1, N), lambda b: (b, 0, 0)),
                  pl.BlockSpec((1, N, 1), lambda b: (b, 0, 0)),
                  pl.BlockSpec((1, N, F), lambda b: (b, 0, 0))],
        out_specs=[pl.BlockSpec((1, _K, F), lambda b: (b, 0, 0)),
                   pl.BlockSpec((1, _K, 1), lambda b: (b, 0, 0)),
                   pl.BlockSpec((1, _K, 1), lambda b: (b, 0, 0))],
        out_shape=[jax.ShapeDtypeStruct((B, _K, F), jnp.float32),
                   jax.ShapeDtypeStruct((B, _K, 1), jnp.int32),
                   jax.ShapeDtypeStruct((B, _K, 1), jnp.int32)],
    )(s_row3, s_col3, H)

    gk = _make_sc_gather(B, N)
    gret = gk(G.reshape(B * N, N), idxg3.reshape(B * _K),
              idxl3.reshape(B * _K))
    return hret, gret.reshape(B, _K, _K)


# speedup vs baseline: 1.2202x; 1.2202x over previous
import functools

import jax
import jax.numpy as jnp
from jax import lax
from jax.experimental import pallas as pl
from jax.experimental.pallas import tpu as pltpu
from jax.experimental.pallas import tpu_sc as plsc

_K = 512
_NC, _NS, _L = 2, 16, 16
_NW = _NC * _NS
_RPW = _K // _NW


def _scores_body(h_ref, w_ref, s_ref):
    h = h_ref[0]
    w = w_ref[...]
    nrm = jnp.sqrt(jnp.sum(w * w))
    s = jnp.dot(h, w, preferred_element_type=jnp.float32)
    s_ref[0] = s / nrm


def _select_body(sr_ref, sc_ref, h_ref, hret_ref, idxl_ref, idxg_ref):
    b = pl.program_id(0)
    s_row = sr_ref[0]
    s_col = sc_ref[0]
    h = h_ref[0]
    n = s_col.shape[0]

    cb = 512
    rank_chunks = []
    for j0 in range(0, n, cb):
        s_rj = lax.slice(s_row, (0, j0), (1, j0 + cb))
        ii = lax.broadcasted_iota(jnp.int32, (n, cb), 0)
        jj = lax.broadcasted_iota(jnp.int32, (n, cb), 1) + j0
        beats = (s_col > s_rj) | ((s_col == s_rj) & (ii < jj))
        rank_chunks.append(
            jnp.sum(beats.astype(jnp.int32), axis=0, keepdims=True))
    ranks = jnp.concatenate(rank_chunks, axis=1)

    rr = lax.broadcasted_iota(jnp.int32, (_K, 1), 0)
    mask = ranks == rr
    jj_row = lax.broadcasted_iota(jnp.int32, (1, n), 1)
    idx = jnp.sum(jnp.where(mask, jj_row, 0), axis=1, keepdims=True)
    ssel = jnp.sum(jnp.where(mask, s_row, 0.0), axis=1, keepdims=True)
    gate = jax.nn.sigmoid(ssel)
    pf = mask.astype(jnp.float32)
    hret = jnp.dot(pf, h, preferred_element_type=jnp.float32,
                   precision=jax.lax.Precision.HIGHEST)
    hret_ref[0] = hret * gate
    idxl_ref[0] = idx
    idxg_ref[0] = idx + b * n


def _make_sc_gather(B, N):
    mesh = plsc.VectorSubcoreMesh(core_axis_name="c", subcore_axis_name="s")

    @functools.partial(
        pl.kernel,
        mesh=mesh,
        out_type=jax.ShapeDtypeStruct((B * _K, _K), jnp.float32),
        scratch_types=[
            pltpu.VMEM((_RPW,), jnp.int32),
            pltpu.VMEM((_K,), jnp.int32),
            pltpu.VMEM((_RPW, N), jnp.float32),
            pltpu.VMEM((_RPW, _K), jnp.float32),
            pltpu.SemaphoreType.DMA,
        ],
        compiler_params=pltpu.CompilerParams(use_tc_tiling_on_sc=True,
                                             needs_layout_passes=False),
    )
    def gather_kernel(gflat, idxg, idxl, out, rowidx_v, colidx_v, rows_v,
                      outb_v, sem):
        wid = lax.axis_index("s") * _NC + lax.axis_index("c")
        for b in range(B):
            base = b * _K + wid * _RPW
            pltpu.sync_copy(idxg.at[pl.ds(base, _RPW)], rowidx_v)
            pltpu.sync_copy(idxl.at[pl.ds(b * _K, _K)], colidx_v)
            pltpu.async_copy(gflat.at[rowidx_v], rows_v, sem).wait()

            def row_body(r, carry):
                rsplat = jnp.broadcast_to(r, (_L,)).astype(jnp.int32)
                for t in range(_K // _L):
                    cols = colidx_v[pl.ds(t * _L, _L)]
                    vals = plsc.load_gather(rows_v, [rsplat, cols])
                    outb_v[r, pl.ds(t * _L, _L)] = vals
                return carry

            lax.fori_loop(0, _RPW, row_body, 0)
            pltpu.sync_copy(outb_v, out.at[pl.ds(base, _RPW)])

    return gather_kernel


def kernel(H, G, score_weight):
    B, N, F = H.shape
    s_col3 = pl.pallas_call(
        _scores_body,
        grid=(B,),
        in_specs=[pl.BlockSpec((1, N, F), lambda b: (b, 0, 0)),
                  pl.BlockSpec((F, 1), lambda b: (0, 0))],
        out_specs=pl.BlockSpec((1, N, 1), lambda b: (b, 0, 0)),
        out_shape=jax.ShapeDtypeStruct((B, N, 1), jnp.float32),
    )(H, score_weight)
    s_row3 = s_col3.reshape(B, 1, N)

    hret, idxl3, idxg3 = pl.pallas_call(
        _select_body,
        grid=(B,),
        in_specs=[pl.BlockSpec((1, 1, N), lambda b: (b, 0, 0)),
                  pl.BlockSpec((1, N, 1), lambda b: (b, 0, 0)),
                  pl.BlockSpec((1, N, F), lambda b: (b, 0, 0))],
        out_specs=[pl.BlockSpec((1, _K, F), lambda b: (b, 0, 0)),
                   pl.BlockSpec((1, _K, 1), lambda b: (b, 0, 0)),
                   pl.BlockSpec((1, _K, 1), lambda b: (b, 0, 0))],
        out_shape=[jax.ShapeDtypeStruct((B, _K, F), jnp.float32),
                   jax.ShapeDtypeStruct((B, _K, 1), jnp.int32),
                   jax.ShapeDtypeStruct((B, _K, 1), jnp.int32)],
    )(s_row3, s_col3, H)

    gk = _make_sc_gather(B, N)
    gret = gk(G.reshape(B * N, N), idxg3.reshape(B * _K),
              idxl3.reshape(B * _K))
    return hret, gret.reshape(B, _K, _K)


# speedup vs baseline: 1.6243x; 1.3312x over previous
import functools

import jax
import jax.numpy as jnp
from jax import lax
from jax.experimental import pallas as pl
from jax.experimental.pallas import tpu as pltpu
from jax.experimental.pallas import tpu_sc as plsc

_K = 512
_NC, _NS, _L = 2, 16, 16
_NW = _NC * _NS
_RPW = _K // _NW


def _scores_body(h_ref, w_ref, s_ref):
    h = h_ref[0]
    w = w_ref[...]
    nrm = jnp.sqrt(jnp.sum(w * w))
    s = jnp.dot(h, w, preferred_element_type=jnp.float32)
    s_ref[0] = s / nrm


def _select_body(sr_ref, sc_ref, h_ref, hret_ref, idxl_ref, idxg_ref):
    b = pl.program_id(0)
    s_row = sr_ref[0]
    s_col = sc_ref[0]
    h = h_ref[0]
    n = s_col.shape[0]

    cb = 512
    rank_chunks = []
    for j0 in range(0, n, cb):
        s_rj = lax.slice(s_row, (0, j0), (1, j0 + cb))
        ii = lax.broadcasted_iota(jnp.int32, (n, cb), 0)
        jj = lax.broadcasted_iota(jnp.int32, (n, cb), 1) + j0
        beats = (s_col > s_rj) | ((s_col == s_rj) & (ii < jj))
        rank_chunks.append(
            jnp.sum(beats.astype(jnp.int32), axis=0, keepdims=True))
    ranks = jnp.concatenate(rank_chunks, axis=1)

    rr = lax.broadcasted_iota(jnp.int32, (_K, 1), 0)
    mask = ranks == rr
    jj_row = lax.broadcasted_iota(jnp.int32, (1, n), 1)
    idx = jnp.sum(jnp.where(mask, jj_row, 0), axis=1, keepdims=True)
    ssel = jnp.sum(jnp.where(mask, s_row, 0.0), axis=1, keepdims=True)
    gate = jax.nn.sigmoid(ssel)
    pf = mask.astype(jnp.float32)
    h_hi = h.astype(jnp.bfloat16).astype(jnp.float32)
    h_lo = h - h_hi
    hret = (jnp.dot(pf, h_hi, preferred_element_type=jnp.float32)
            + jnp.dot(pf, h_lo, preferred_element_type=jnp.float32))
    hret_ref[0] = hret * gate
    idxl_ref[0] = idx
    idxg_ref[0] = idx + b * n


def _make_sc_gather(B, N):
    mesh = plsc.VectorSubcoreMesh(core_axis_name="c", subcore_axis_name="s")

    @functools.partial(
        pl.kernel,
        mesh=mesh,
        out_type=jax.ShapeDtypeStruct((B * _K, _K), jnp.float32),
        scratch_types=[
            pltpu.VMEM((B * _K,), jnp.int32),
            pltpu.VMEM((B * _K,), jnp.int32),
            pltpu.VMEM((2, _RPW, N), jnp.float32),
            pltpu.VMEM((2, _RPW, _K), jnp.float32),
            pltpu.SemaphoreType.DMA((2,)),
            pltpu.SemaphoreType.DMA((2,)),
        ],
        compiler_params=pltpu.CompilerParams(use_tc_tiling_on_sc=True,
                                             needs_layout_passes=False),
    )
    def gather_kernel(gflat, idxg, idxl, out, idxg_v, idxl_v, rows_v,
                      outb_v, gsem, osem):
        wid = lax.axis_index("s") * _NC + lax.axis_index("c")
        pltpu.sync_copy(idxg, idxg_v)
        pltpu.sync_copy(idxl, idxl_v)

        def start_gather(b):
            rowvec = idxg_v[pl.ds(b * _K + wid * _RPW, _RPW)]
            return pltpu.async_copy(gflat.at[rowvec], rows_v.at[b % 2],
                                    gsem.at[b % 2])

        gather_handles = [start_gather(0), None]
        out_handles = [None, None]
        for b in range(B):
            cur = b % 2
            if b + 1 < B:
                gather_handles[1 - cur] = start_gather(b + 1)
            gather_handles[cur].wait()
            if out_handles[cur] is not None:
                out_handles[cur].wait()
            rv = rows_v.at[cur]
            ov = outb_v.at[cur]

            def row_body(r, carry, rv=rv, ov=ov, b=b):
                rsplat = jnp.broadcast_to(r, (_L,)).astype(jnp.int32)
                for t in range(_K // _L):
                    cols = idxl_v[pl.ds(b * _K + t * _L, _L)]
                    vals = plsc.load_gather(rv, [rsplat, cols])
                    ov[r, pl.ds(t * _L, _L)] = vals
                return carry

            lax.fori_loop(0, _RPW, row_body, 0)
            base = b * _K + wid * _RPW
            out_handles[cur] = pltpu.async_copy(
                ov, out.at[pl.ds(base, _RPW)], osem.at[cur])
        for h in out_handles:
            if h is not None:
                h.wait()

    return gather_kernel


def kernel(H, G, score_weight):
    B, N, F = H.shape
    s_col3 = pl.pallas_call(
        _scores_body,
        grid=(B,),
        in_specs=[pl.BlockSpec((1, N, F), lambda b: (b, 0, 0)),
                  pl.BlockSpec((F, 1), lambda b: (0, 0))],
        out_specs=pl.BlockSpec((1, N, 1), lambda b: (b, 0, 0)),
        out_shape=jax.ShapeDtypeStruct((B, N, 1), jnp.float32),
    )(H, score_weight)
    s_row3 = s_col3.reshape(B, 1, N)

    hret, idxl3, idxg3 = pl.pallas_call(
        _select_body,
        grid=(B,),
        in_specs=[pl.BlockSpec((1, 1, N), lambda b: (b, 0, 0)),
                  pl.BlockSpec((1, N, 1), lambda b: (b, 0, 0)),
                  pl.BlockSpec((1, N, F), lambda b: (b, 0, 0))],
        out_specs=[pl.BlockSpec((1, _K, F), lambda b: (b, 0, 0)),
                   pl.BlockSpec((1, _K, 1), lambda b: (b, 0, 0)),
                   pl.BlockSpec((1, _K, 1), lambda b: (b, 0, 0))],
        out_shape=[jax.ShapeDtypeStruct((B, _K, F), jnp.float32),
                   jax.ShapeDtypeStruct((B, _K, 1), jnp.int32),
                   jax.ShapeDtypeStruct((B, _K, 1), jnp.int32)],
    )(s_row3, s_col3, H)

    gk = _make_sc_gather(B, N)
    gret = gk(G.reshape(B * N, N), idxg3.reshape(B * _K),
              idxl3.reshape(B * _K))
    return hret, gret.reshape(B, _K, _K)


# speedup vs baseline: 2.0900x; 1.2867x over previous
import functools

import jax
import jax.numpy as jnp
from jax import lax
from jax.experimental import pallas as pl
from jax.experimental.pallas import tpu as pltpu
from jax.experimental.pallas import tpu_sc as plsc

_K = 512
_NC, _NS, _L = 2, 16, 16
_NW = _NC * _NS
_RPW = _K // _NW


def _scores_body(h_ref, w_ref, s_ref):
    h = h_ref[0]
    w = w_ref[...]
    nrm = jnp.sqrt(jnp.sum(w * w))
    s = jnp.dot(h, w, preferred_element_type=jnp.float32)
    s_ref[0] = s / nrm


def _select_body(sr_ref, sc_ref, h_ref, hret_ref, idxl_ref, idxg_ref):
    b = pl.program_id(0)
    s_row = sr_ref[0]
    s_col = sc_ref[0]
    h = h_ref[0]
    n = s_col.shape[0]

    cb = 512
    rank_chunks = []
    for j0 in range(0, n, cb):
        s_rj = lax.slice(s_row, (0, j0), (1, j0 + cb))
        ii = lax.broadcasted_iota(jnp.int32, (n, cb), 0)
        jj = lax.broadcasted_iota(jnp.int32, (n, cb), 1) + j0
        beats = (s_col > s_rj) | ((s_col == s_rj) & (ii < jj))
        rank_chunks.append(
            jnp.sum(beats.astype(jnp.int32), axis=0, keepdims=True))
    ranks = jnp.concatenate(rank_chunks, axis=1)

    rr = lax.broadcasted_iota(jnp.int32, (_K, 1), 0)
    mask = ranks == rr
    jj_row = lax.broadcasted_iota(jnp.int32, (1, n), 1)
    idx = jnp.sum(jnp.where(mask, jj_row, 0), axis=1, keepdims=True)
    ssel = jnp.sum(jnp.where(mask, s_row, 0.0), axis=1, keepdims=True)
    gate = jax.nn.sigmoid(ssel)
    pf = mask.astype(jnp.float32)
    h_hi = h.astype(jnp.bfloat16).astype(jnp.float32)
    h_lo = h - h_hi
    hret = (jnp.dot(pf, h_hi, preferred_element_type=jnp.float32)
            + jnp.dot(pf, h_lo, preferred_element_type=jnp.float32))
    hret_ref[0] = hret * gate
    idxl_ref[0] = idx
    idxg_ref[0] = idx + b * n


def _make_sc_gather(B, N):
    mesh = plsc.VectorSubcoreMesh(core_axis_name="c", subcore_axis_name="s")

    @functools.partial(
        pl.kernel,
        mesh=mesh,
        out_type=jax.ShapeDtypeStruct((B * _K, _K), jnp.float32),
        scratch_types=[
            pltpu.VMEM((B * _K,), jnp.int32),
            pltpu.VMEM((B * _K,), jnp.int32),
            pltpu.VMEM((3, _RPW, N), jnp.float32),
            pltpu.VMEM((2, _RPW, _K), jnp.float32),
            pltpu.SemaphoreType.DMA((3,)),
            pltpu.SemaphoreType.DMA((2,)),
            pltpu.SemaphoreType.DMA((2,)),
        ],
        compiler_params=pltpu.CompilerParams(use_tc_tiling_on_sc=True,
                                             needs_layout_passes=False),
    )
    def gather_kernel(gflat, idxg, idxl, out, idxg_v, idxl_v, rows_v,
                      outb_v, gsem, osem, isem):
        wid = lax.axis_index("s") * _NC + lax.axis_index("c")
        hg = pltpu.async_copy(idxg, idxg_v, isem.at[0])
        hl = pltpu.async_copy(idxl, idxl_v, isem.at[1])
        hg.wait()

        nsub, spr = 2, _RPW // 2

        def start_gather(b):
            sl = b % 3
            hs = []
            for q in range(nsub):
                iref = idxg_v.at[pl.ds(b * _K + wid * _RPW + q * spr, spr)]
                hs.append(pltpu.async_copy(
                    gflat.at[iref], rows_v.at[sl, pl.ds(q * spr, spr)],
                    gsem.at[sl]))
            return hs

        gather_handles = [start_gather(0), None, None]
        if B > 1:
            gather_handles[1] = start_gather(1)
        hl.wait()
        out_handles = [None, None]
        for b in range(B):
            sl = b % 3
            if b + 2 < B:
                gather_handles[(b + 2) % 3] = start_gather(b + 2)
            for h in gather_handles[sl]:
                h.wait()
            ocur = b % 2
            if out_handles[ocur] is not None:
                out_handles[ocur].wait()
            rv = rows_v.at[sl]
            ov = outb_v.at[ocur]
            cols = [idxl_v[pl.ds(b * _K + t * _L, _L)]
                    for t in range(_K // _L)]

            def row_body(r, carry, rv=rv, ov=ov, cols=cols):
                rsplat = jnp.broadcast_to(r, (_L,)).astype(jnp.int32)
                for t in range(_K // _L):
                    vals = plsc.load_gather(rv, [rsplat, cols[t]])
                    ov[r, pl.ds(t * _L, _L)] = vals
                return carry

            lax.fori_loop(0, _RPW, row_body, 0)
            base = b * _K + wid * _RPW
            out_handles[ocur] = pltpu.async_copy(
                ov, out.at[pl.ds(base, _RPW)], osem.at[ocur])
        for h in out_handles:
            if h is not None:
                h.wait()

    return gather_kernel


def kernel(H, G, score_weight):
    B, N, F = H.shape
    s_col3 = pl.pallas_call(
        _scores_body,
        grid=(B,),
        in_specs=[pl.BlockSpec((1, N, F), lambda b: (b, 0, 0)),
                  pl.BlockSpec((F, 1), lambda b: (0, 0))],
        out_specs=pl.BlockSpec((1, N, 1), lambda b: (b, 0, 0)),
        out_shape=jax.ShapeDtypeStruct((B, N, 1), jnp.float32),
    )(H, score_weight)
    s_row3 = s_col3.reshape(B, 1, N)

    hret, idxl3, idxg3 = pl.pallas_call(
        _select_body,
        grid=(B,),
        in_specs=[pl.BlockSpec((1, 1, N), lambda b: (b, 0, 0)),
                  pl.BlockSpec((1, N, 1), lambda b: (b, 0, 0)),
                  pl.BlockSpec((1, N, F), lambda b: (b, 0, 0))],
        out_specs=[pl.BlockSpec((1, _K, F), lambda b: (b, 0, 0)),
                   pl.BlockSpec((1, _K, 1), lambda b: (b, 0, 0)),
                   pl.BlockSpec((1, _K, 1), lambda b: (b, 0, 0))],
        out_shape=[jax.ShapeDtypeStruct((B, _K, F), jnp.float32),
                   jax.ShapeDtypeStruct((B, _K, 1), jnp.int32),
                   jax.ShapeDtypeStruct((B, _K, 1), jnp.int32)],
    )(s_row3, s_col3, H)

    gk = _make_sc_gather(B, N)
    gret = gk(G.reshape(B * N, N), idxg3.reshape(B * _K),
              idxl3.reshape(B * _K))
    return hret, gret.reshape(B, _K, _K)
